# baseline (speedup 1.0000x reference)
import functools

import jax
import jax.numpy as jnp
from jax import lax
from jax.experimental import pallas as pl
from jax.experimental.pallas import tpu as pltpu
from jax.experimental.pallas import tpu_sc as plsc

NC = 2
NS = 16
L = 16
NW = NC * NS

VOCAB = 1000000
D = 64
B = 16384
NCTX = 20

B_PER_W = B // NW
GROUP_B = L
GROUPS = B_PER_W // GROUP_B
GROUP_ROWS = GROUP_B * NCTX
CHUNKS = ((0, 128), (128, 128), (256, 64))


def _rsqrt(s):
    i = plsc.bitcast(s, jnp.int32)
    i = 0x5F3759DF - lax.shift_right_arithmetic(i, 1)
    y = plsc.bitcast(i, jnp.float32)
    h = s * 0.5
    for _ in range(3):
        y = y * (1.5 - h * y * y)
    return y


def _sc_body(centers, contexts, cidx, ctxidx, negidx, out,
             cidx_v, crows, ct, outv,
             ci0, ci1, ni0, ni1, cr0, cr1, nr0, nr1,
             sem_c, sem_x0, sem_x1, sem_n0, sem_n1):
    wid = lax.axis_index("s") * NC + lax.axis_index("c")
    w_base = wid * B_PER_W

    iota = lax.iota(jnp.int32, L)
    iota_nctx = iota * NCTX

    def fire(g, ci, ni, cr, nr, sem_x, sem_n):
        off = (w_base + g * GROUP_B) * NCTX
        pltpu.sync_copy(ctxidx.at[pl.ds(off, GROUP_ROWS)], ci)
        pltpu.sync_copy(negidx.at[pl.ds(off, GROUP_ROWS)], ni)
        for s, n in CHUNKS:
            pltpu.async_copy(contexts.at[ci.at[pl.ds(s, n)]],
                             cr.at[pl.ds(s, n)], sem_x)
            pltpu.async_copy(contexts.at[ni.at[pl.ds(s, n)]],
                             nr.at[pl.ds(s, n)], sem_n)

    def drain(ci, ni, cr, nr, sem_x, sem_n):
        for s, n in CHUNKS:
            pltpu.make_async_copy(contexts.at[ci.at[pl.ds(s, n)]],
                                  cr.at[pl.ds(s, n)], sem_x).wait()
            pltpu.make_async_copy(contexts.at[ni.at[pl.ds(s, n)]],
                                  nr.at[pl.ds(s, n)], sem_n).wait()

    pltpu.sync_copy(cidx.at[pl.ds(w_base, B_PER_W)], cidx_v)
    for k in range(B_PER_W // 128):
        pltpu.async_copy(centers.at[cidx_v.at[pl.ds(k * 128, 128)]],
                         crows.at[pl.ds(k * 128, 128)], sem_c)
    fire(0, ci0, ni0, cr0, nr0, sem_x0, sem_n0)
    for k in range(B_PER_W // 128):
        pltpu.make_async_copy(centers.at[cidx_v.at[pl.ds(k * 128, 128)]],
                              crows.at[pl.ds(k * 128, 128)], sem_c).wait()

    def compute(g, cr, nr, acc):
        base = g * GROUP_B
        rowc = iota + base

        def tbody(d, cc):
            dv = jnp.broadcast_to(d, (L,))
            cv = plsc.load_gather(crows, [rowc, dv])
            ct[d] = cv
            return cc + cv * cv
        cc = lax.fori_loop(0, D, tbody, jnp.zeros((L,), jnp.float32))

        for j in range(NCTX):
            rowj = iota_nctx + j

            def dbody(d, carry):
                dotp, xxp, dotn, xxn = carry
                dv = jnp.broadcast_to(d, (L,))
                cv = ct[d]
                xp = plsc.load_gather(cr, [rowj, dv])
                xn = plsc.load_gather(nr, [rowj, dv])
                return (dotp + cv * xp, xxp + xp * xp,
                        dotn + cv * xn, xxn + xn * xn)

            z = jnp.zeros((L,), jnp.float32)
            dotp, xxp, dotn, xxn = lax.fori_loop(0, D, dbody, (z, z, z, z))
            cosp = dotp * _rsqrt(jnp.maximum(cc * xxp, 1e-16))
            cosn = dotn * _rsqrt(jnp.maximum(cc * xxn, 1e-16))
            acc = acc + ((1.0 - cosp) + jnp.maximum(cosn, 0.0))
        return acc

    def pairbody(t, acc):
        g0 = 2 * t
        g1 = g0 + 1
        fire(g1, ci1, ni1, cr1, nr1, sem_x1, sem_n1)
        drain(ci0, ni0, cr0, nr0, sem_x0, sem_n0)
        acc = compute(g0, cr0, nr0, acc)

        @pl.when(t < GROUPS // 2 - 1)
        def _():
            fire(g1 + 1, ci0, ni0, cr0, nr0, sem_x0, sem_n0)
        drain(ci1, ni1, cr1, nr1, sem_x1, sem_n1)
        acc = compute(g1, cr1, nr1, acc)
        return acc

    acc = lax.fori_loop(0, GROUPS // 2, pairbody,
                        jnp.zeros((L,), jnp.float32))

    outv[...] = acc
    pltpu.sync_copy(outv, out.at[wid])


@jax.jit
def _sc_partials(centers, contexts, cidx, ctxidx, negidx):
    mesh = plsc.VectorSubcoreMesh(core_axis_name="c", subcore_axis_name="s")
    f = functools.partial(
        pl.kernel,
        out_type=jax.ShapeDtypeStruct((NW, L), jnp.float32),
        mesh=mesh,
        compiler_params=pltpu.CompilerParams(needs_layout_passes=False,
                                             use_tc_tiling_on_sc=False),
        scratch_types=[
            pltpu.VMEM((B_PER_W,), jnp.int32),
            pltpu.VMEM((B_PER_W, D), jnp.float32),
            pltpu.VMEM((D, L), jnp.float32),
            pltpu.VMEM((L,), jnp.float32),
            pltpu.VMEM((GROUP_ROWS,), jnp.int32),
            pltpu.VMEM((GROUP_ROWS,), jnp.int32),
            pltpu.VMEM((GROUP_ROWS,), jnp.int32),
            pltpu.VMEM((GROUP_ROWS,), jnp.int32),
            pltpu.VMEM((GROUP_ROWS, D), jnp.float32),
            pltpu.VMEM((GROUP_ROWS, D), jnp.float32),
            pltpu.VMEM((GROUP_ROWS, D), jnp.float32),
            pltpu.VMEM((GROUP_ROWS, D), jnp.float32),
            pltpu.SemaphoreType.DMA,
            pltpu.SemaphoreType.DMA,
            pltpu.SemaphoreType.DMA,
            pltpu.SemaphoreType.DMA,
            pltpu.SemaphoreType.DMA,
        ],
    )(_sc_body)
    return f(centers, contexts, cidx, ctxidx, negidx)


def _tc_reduce_body(x_ref, o_ref):
    s = jnp.sum(x_ref[...]) * (1.0 / (B * NCTX))
    o_ref[...] = jnp.reshape(s, (1, 1))


@jax.jit
def _tc_reduce(partials):
    return pl.pallas_call(
        _tc_reduce_body,
        out_shape=jax.ShapeDtypeStruct((1, 1), jnp.float32),
    )(partials)


def kernel(centers, contexts, center_idxs, context_idxs, neg_idxs):
    cidx = center_idxs.astype(jnp.int32)
    ctxidx = context_idxs.astype(jnp.int32).reshape(-1)
    negidx = neg_idxs.astype(jnp.int32)
    partials = _sc_partials(centers, contexts, cidx, ctxidx, negidx)
    return _tc_reduce(partials)[0, 0]


# speedup vs baseline: 1.0104x; 1.0104x over previous
import functools

import jax
import jax.numpy as jnp
from jax import lax
from jax.experimental import pallas as pl
from jax.experimental.pallas import tpu as pltpu
from jax.experimental.pallas import tpu_sc as plsc

NC = 2
NS = 16
L = 16
NW = NC * NS

VOCAB = 1000000
D = 64
B = 16384
NCTX = 20

B_PER_W = B // NW
GROUP_B = L
GROUPS = B_PER_W // GROUP_B
GROUP_ROWS = GROUP_B * NCTX
CHUNKS = ((0, 128), (128, 128), (256, 64))


def _rsqrt(s):
    i = plsc.bitcast(s, jnp.int32)
    i = 0x5F3759DF - lax.shift_right_arithmetic(i, 1)
    y = plsc.bitcast(i, jnp.float32)
    h = s * 0.5
    for _ in range(3):
        y = y * (1.5 - h * y * y)
    return y


def _sc_body(centers, contexts, cidx, ctxidx, negidx, out,
             cidx_v, crows, ct, outv,
             ci0, ci1, ni0, ni1, cr0, cr1, nr0, nr1,
             sem_c, sem_x0, sem_x1, sem_n0, sem_n1):
    wid = lax.axis_index("s") * NC + lax.axis_index("c")
    w_base = wid * B_PER_W

    iota = lax.iota(jnp.int32, L)
    iota_nctx = iota * NCTX

    def fire(g, ci, ni, cr, nr, sem_x, sem_n):
        off = (w_base + g * GROUP_B) * NCTX
        pltpu.sync_copy(ctxidx.at[pl.ds(off, GROUP_ROWS)], ci)
        pltpu.sync_copy(negidx.at[pl.ds(off, GROUP_ROWS)], ni)
        for s, n in CHUNKS:
            pltpu.async_copy(contexts.at[ci.at[pl.ds(s, n)]],
                             cr.at[pl.ds(s, n)], sem_x)
            pltpu.async_copy(contexts.at[ni.at[pl.ds(s, n)]],
                             nr.at[pl.ds(s, n)], sem_n)

    def drain(ci, ni, cr, nr, sem_x, sem_n):
        for s, n in CHUNKS:
            pltpu.make_async_copy(contexts.at[ci.at[pl.ds(s, n)]],
                                  cr.at[pl.ds(s, n)], sem_x).wait()
            pltpu.make_async_copy(contexts.at[ni.at[pl.ds(s, n)]],
                                  nr.at[pl.ds(s, n)], sem_n).wait()

    pltpu.sync_copy(cidx.at[pl.ds(w_base, B_PER_W)], cidx_v)
    for k in range(B_PER_W // 128):
        pltpu.async_copy(centers.at[cidx_v.at[pl.ds(k * 128, 128)]],
                         crows.at[pl.ds(k * 128, 128)], sem_c)
    fire(0, ci0, ni0, cr0, nr0, sem_x0, sem_n0)
    for k in range(B_PER_W // 128):
        pltpu.make_async_copy(centers.at[cidx_v.at[pl.ds(k * 128, 128)]],
                              crows.at[pl.ds(k * 128, 128)], sem_c).wait()

    dconsts = [jnp.full((L,), d, jnp.int32) for d in range(D)]

    def compute(g, cr, nr, acc):
        base = g * GROUP_B
        rowc = iota + base

        def tbody(t, carry):
            cc0, cc1 = carry
            d0 = t * 8
            for u in range(8):
                cv = plsc.load_gather(crows, [rowc, dconsts[0] + (d0 + u)])
                ct[d0 + u] = cv
                if u % 2 == 0:
                    cc0 = cc0 + cv * cv
                else:
                    cc1 = cc1 + cv * cv
            return (cc0, cc1)
        z = jnp.zeros((L,), jnp.float32)
        cc0, cc1 = lax.fori_loop(0, D // 8, tbody, (z, z))
        cc = cc0 + cc1

        def jbody(j, acc):
            rowj = iota_nctx + j
            dp = [z, z]
            xxpa = [z, z]
            dn = [z, z]
            xxna = [z, z]
            for d in range(D):
                k = d & 1
                cv = ct[d]
                xp = plsc.load_gather(cr, [rowj, dconsts[d]])
                xn = plsc.load_gather(nr, [rowj, dconsts[d]])
                dp[k] = dp[k] + cv * xp
                xxpa[k] = xxpa[k] + xp * xp
                dn[k] = dn[k] + cv * xn
                xxna[k] = xxna[k] + xn * xn
            dotp = dp[0] + dp[1]
            xxp = xxpa[0] + xxpa[1]
            dotn = dn[0] + dn[1]
            xxn = xxna[0] + xxna[1]
            cosp = dotp * _rsqrt(jnp.maximum(cc * xxp, 1e-16))
            cosn = dotn * _rsqrt(jnp.maximum(cc * xxn, 1e-16))
            return acc + ((1.0 - cosp) + jnp.maximum(cosn, 0.0))

        return lax.fori_loop(0, NCTX, jbody, acc)

    def pairbody(t, acc):
        g0 = 2 * t
        g1 = g0 + 1
        fire(g1, ci1, ni1, cr1, nr1, sem_x1, sem_n1)
        drain(ci0, ni0, cr0, nr0, sem_x0, sem_n0)
        acc = compute(g0, cr0, nr0, acc)

        @pl.when(t < GROUPS // 2 - 1)
        def _():
            fire(g1 + 1, ci0, ni0, cr0, nr0, sem_x0, sem_n0)
        drain(ci1, ni1, cr1, nr1, sem_x1, sem_n1)
        acc = compute(g1, cr1, nr1, acc)
        return acc

    acc = lax.fori_loop(0, GROUPS // 2, pairbody,
                        jnp.zeros((L,), jnp.float32))

    outv[...] = acc
    pltpu.sync_copy(outv, out.at[wid])


@jax.jit
def _sc_partials(centers, contexts, cidx, ctxidx, negidx):
    mesh = plsc.VectorSubcoreMesh(core_axis_name="c", subcore_axis_name="s")
    f = functools.partial(
        pl.kernel,
        out_type=jax.ShapeDtypeStruct((NW, L), jnp.float32),
        mesh=mesh,
        compiler_params=pltpu.CompilerParams(needs_layout_passes=False,
                                             use_tc_tiling_on_sc=False),
        scratch_types=[
            pltpu.VMEM((B_PER_W,), jnp.int32),
            pltpu.VMEM((B_PER_W, D), jnp.float32),
            pltpu.VMEM((D, L), jnp.float32),
            pltpu.VMEM((L,), jnp.float32),
            pltpu.VMEM((GROUP_ROWS,), jnp.int32),
            pltpu.VMEM((GROUP_ROWS,), jnp.int32),
            pltpu.VMEM((GROUP_ROWS,), jnp.int32),
            pltpu.VMEM((GROUP_ROWS,), jnp.int32),
            pltpu.VMEM((GROUP_ROWS, D), jnp.float32),
            pltpu.VMEM((GROUP_ROWS, D), jnp.float32),
            pltpu.VMEM((GROUP_ROWS, D), jnp.float32),
            pltpu.VMEM((GROUP_ROWS, D), jnp.float32),
            pltpu.SemaphoreType.DMA,
            pltpu.SemaphoreType.DMA,
            pltpu.SemaphoreType.DMA,
            pltpu.SemaphoreType.DMA,
            pltpu.SemaphoreType.DMA,
        ],
    )(_sc_body)
    return f(centers, contexts, cidx, ctxidx, negidx)


def _tc_reduce_body(x_ref, o_ref):
    s = jnp.sum(x_ref[...]) * (1.0 / (B * NCTX))
    o_ref[...] = jnp.reshape(s, (1, 1))


@jax.jit
def _tc_reduce(partials):
    return pl.pallas_call(
        _tc_reduce_body,
        out_shape=jax.ShapeDtypeStruct((1, 1), jnp.float32),
    )(partials)


def kernel(centers, contexts, center_idxs, context_idxs, neg_idxs):
    cidx = center_idxs.astype(jnp.int32)
    ctxidx = context_idxs.astype(jnp.int32).reshape(-1)
    negidx = neg_idxs.astype(jnp.int32)
    partials = _sc_partials(centers, contexts, cidx, ctxidx, negidx)
    return _tc_reduce(partials)[0, 0]


# speedup vs baseline: 1.4472x; 1.4323x over previous
import functools

import jax
import jax.numpy as jnp
from jax import lax
from jax.experimental import pallas as pl
from jax.experimental.pallas import tpu as pltpu
from jax.experimental.pallas import tpu_sc as plsc

NC = 2
NS = 16
L = 16
NW = NC * NS

VOCAB = 1000000
D = 64
B = 16384
NCTX = 20

B_PER_W = B // NW
GROUP_B = L
GROUPS = B_PER_W // GROUP_B
GROUP_ROWS = GROUP_B * NCTX
CHUNKS = ((0, 128), (128, 128), (256, 64))


def _rsqrt(s):
    i = plsc.bitcast(s, jnp.int32)
    i = 0x5F3759DF - lax.shift_right_arithmetic(i, 1)
    y = plsc.bitcast(i, jnp.float32)
    h = s * 0.5
    for _ in range(3):
        y = y * (1.5 - h * y * y)
    return y


def _sc_body(centers, contexts, cidx, ctxidx, negidx, out,
             cidx_v, crows, ct, outv,
             ci0, ci1, ni0, ni1, cr0, cr1, nr0, nr1,
             sem_c, sem_x0, sem_x1, sem_n0, sem_n1):
    wid = lax.axis_index("s") * NC + lax.axis_index("c")
    w_base = wid * B_PER_W

    iota = lax.iota(jnp.int32, L)
    iota_nctx = iota * NCTX

    def fire(g, ci, ni, cr, nr, sem_x, sem_n):
        off = (w_base + g * GROUP_B) * NCTX
        pltpu.sync_copy(ctxidx.at[pl.ds(off, GROUP_ROWS)], ci)
        pltpu.sync_copy(negidx.at[pl.ds(off, GROUP_ROWS)], ni)
        for s, n in CHUNKS:
            pltpu.async_copy(contexts.at[ci.at[pl.ds(s, n)]],
                             cr.at[pl.ds(s, n)], sem_x)
            pltpu.async_copy(contexts.at[ni.at[pl.ds(s, n)]],
                             nr.at[pl.ds(s, n)], sem_n)

    def drain(ci, ni, cr, nr, sem_x, sem_n):
        for s, n in CHUNKS:
            pltpu.make_async_copy(contexts.at[ci.at[pl.ds(s, n)]],
                                  cr.at[pl.ds(s, n)], sem_x).wait()
            pltpu.make_async_copy(contexts.at[ni.at[pl.ds(s, n)]],
                                  nr.at[pl.ds(s, n)], sem_n).wait()

    pltpu.sync_copy(cidx.at[pl.ds(w_base, B_PER_W)], cidx_v)
    for k in range(B_PER_W // 128):
        pltpu.async_copy(centers.at[cidx_v.at[pl.ds(k * 128, 128)]],
                         crows.at[pl.ds(k * 128, 128)], sem_c)
    fire(0, ci0, ni0, cr0, nr0, sem_x0, sem_n0)
    for k in range(B_PER_W // 128):
        pltpu.make_async_copy(centers.at[cidx_v.at[pl.ds(k * 128, 128)]],
                              crows.at[pl.ds(k * 128, 128)], sem_c).wait()

    dperm = [jnp.bitwise_and(iota + d, D - 1) for d in range(D)]

    def dperm_dyn(dd):
        return jnp.bitwise_and(iota + dd, D - 1)

    def compute(g, cr, nr, acc):
        base = g * GROUP_B
        rowc = iota + base

        def tbody(t, carry):
            cc0, cc1 = carry
            d0 = t * 8
            for u in range(8):
                cv = plsc.load_gather(crows, [rowc, dperm_dyn(d0 + u)])
                ct[d0 + u] = cv
                if u % 2 == 0:
                    cc0 = cc0 + cv * cv
                else:
                    cc1 = cc1 + cv * cv
            return (cc0, cc1)
        z = jnp.zeros((L,), jnp.float32)
        cc0, cc1 = lax.fori_loop(0, D // 8, tbody, (z, z))
        cc = cc0 + cc1

        def jbody(j, acc):
            rowj = iota_nctx + j
            dp = [z, z]
            xxpa = [z, z]
            dn = [z, z]
            xxna = [z, z]
            for d in range(D):
                k = d & 1
                cv = ct[d]
                xp = plsc.load_gather(cr, [rowj, dperm[d]])
                xn = plsc.load_gather(nr, [rowj, dperm[d]])
                dp[k] = dp[k] + cv * xp
                xxpa[k] = xxpa[k] + xp * xp
                dn[k] = dn[k] + cv * xn
                xxna[k] = xxna[k] + xn * xn
            dotp = dp[0] + dp[1]
            xxp = xxpa[0] + xxpa[1]
            dotn = dn[0] + dn[1]
            xxn = xxna[0] + xxna[1]
            cosp = dotp * _rsqrt(jnp.maximum(cc * xxp, 1e-16))
            cosn = dotn * _rsqrt(jnp.maximum(cc * xxn, 1e-16))
            return acc + ((1.0 - cosp) + jnp.maximum(cosn, 0.0))

        return lax.fori_loop(0, NCTX, jbody, acc)

    def pairbody(t, acc):
        g0 = 2 * t
        g1 = g0 + 1
        fire(g1, ci1, ni1, cr1, nr1, sem_x1, sem_n1)
        drain(ci0, ni0, cr0, nr0, sem_x0, sem_n0)
        acc = compute(g0, cr0, nr0, acc)

        @pl.when(t < GROUPS // 2 - 1)
        def _():
            fire(g1 + 1, ci0, ni0, cr0, nr0, sem_x0, sem_n0)
        drain(ci1, ni1, cr1, nr1, sem_x1, sem_n1)
        acc = compute(g1, cr1, nr1, acc)
        return acc

    acc = lax.fori_loop(0, GROUPS // 2, pairbody,
                        jnp.zeros((L,), jnp.float32))

    outv[...] = acc
    pltpu.sync_copy(outv, out.at[wid])


@jax.jit
def _sc_partials(centers, contexts, cidx, ctxidx, negidx):
    mesh = plsc.VectorSubcoreMesh(core_axis_name="c", subcore_axis_name="s")
    f = functools.partial(
        pl.kernel,
        out_type=jax.ShapeDtypeStruct((NW, L), jnp.float32),
        mesh=mesh,
        compiler_params=pltpu.CompilerParams(needs_layout_passes=False,
                                             use_tc_tiling_on_sc=False),
        scratch_types=[
            pltpu.VMEM((B_PER_W,), jnp.int32),
            pltpu.VMEM((B_PER_W, D), jnp.float32),
            pltpu.VMEM((D, L), jnp.float32),
            pltpu.VMEM((L,), jnp.float32),
            pltpu.VMEM((GROUP_ROWS,), jnp.int32),
            pltpu.VMEM((GROUP_ROWS,), jnp.int32),
            pltpu.VMEM((GROUP_ROWS,), jnp.int32),
            pltpu.VMEM((GROUP_ROWS,), jnp.int32),
            pltpu.VMEM((GROUP_ROWS, D), jnp.float32),
            pltpu.VMEM((GROUP_ROWS, D), jnp.float32),
            pltpu.VMEM((GROUP_ROWS, D), jnp.float32),
            pltpu.VMEM((GROUP_ROWS, D), jnp.float32),
            pltpu.SemaphoreType.DMA,
            pltpu.SemaphoreType.DMA,
            pltpu.SemaphoreType.DMA,
            pltpu.SemaphoreType.DMA,
            pltpu.SemaphoreType.DMA,
        ],
    )(_sc_body)
    return f(centers, contexts, cidx, ctxidx, negidx)


def _tc_reduce_body(x_ref, o_ref):
    s = jnp.sum(x_ref[...]) * (1.0 / (B * NCTX))
    o_ref[...] = jnp.reshape(s, (1, 1))


@jax.jit
def _tc_reduce(partials):
    return pl.pallas_call(
        _tc_reduce_body,
        out_shape=jax.ShapeDtypeStruct((1, 1), jnp.float32),
    )(partials)


def kernel(centers, contexts, center_idxs, context_idxs, neg_idxs):
    cidx = center_idxs.astype(jnp.int32)
    ctxidx = context_idxs.astype(jnp.int32).reshape(-1)
    negidx = neg_idxs.astype(jnp.int32)
    partials = _sc_partials(centers, contexts, cidx, ctxidx, negidx)
    return _tc_reduce(partials)[0, 0]
